# baseline (speedup 1.0000x reference)
import jax
import jax.numpy as jnp
from jax import lax
from jax.experimental import pallas as pl

_H = 64
_W = 64
_A = 9
_STRIDE = 8.0
_BOX = 32.0
_LN2 = 0.6931471805599453


def _gen_body(o1_ref, o2_ref):
    r = lax.broadcasted_iota(jnp.int32, (_H * _W, _A * 4), 0)
    j = lax.broadcasted_iota(jnp.int32, (_H * _W, _A * 4), 1)
    c = j % 4
    a = j // 4
    s = (a % 3).astype(jnp.float32)
    t = (a // 3).astype(jnp.float32)
    cx = ((r % _W).astype(jnp.float32) + 0.5) * _STRIDE
    cy = ((r // _W).astype(jnp.float32) + 0.5) * _STRIDE
    bw = _BOX * jnp.exp2(s * (1.0 / 3.0) + (t - 1.0) * 0.5)
    bh = _BOX * jnp.exp2(s * (1.0 / 3.0) - (t - 1.0) * 0.5)
    o1_ref[...] = jnp.where(
        c == 0, cx, jnp.where(c == 1, cy, jnp.where(c == 2, bw, bh)))
    o2_ref[...] = jnp.where(
        c == 0, cx - bw * 0.5,
        jnp.where(c == 1, cy - bh * 0.5,
                  jnp.where(c == 2, cx + bw * 0.5, cy + bh * 0.5)))


def kernel(features):
    del features
    o1, o2 = pl.pallas_call(
        _gen_body,
        out_shape=(
            jax.ShapeDtypeStruct((_H * _W, _A * 4), jnp.float32),
            jax.ShapeDtypeStruct((_H * _W, _A * 4), jnp.float32),
        ),
    )()
    return o1.reshape(_H * _W * _A, 4), o2.reshape(_H * _W * _A, 4)
